# baseline (device time: 93067 ns/iter reference)
import numpy as np
import jax
import jax.numpy as jnp
from jax import lax
from jax.experimental import pallas as pl
from jax.experimental.pallas import tpu as pltpu

N_DEV = 8
B, SQ, D = 1, 1024, 1024
HQ, DH = 8, 128
HR = SQ // 2
CH = HR // N_DEV
CHUNK = SQ // N_DEV
SCALE = 0.08838834764831843
ROFFS = (0, 256, 384)


def _rope_consts():
    inv = 1.0 / (10000.0 ** (np.arange(0, DH, 2) / DH))
    pos = np.arange(SQ)[:, None] * inv[None, :]
    cos = np.repeat(np.cos(pos), 2, axis=-1)
    sin = np.repeat(np.sin(pos), 2, axis=-1)
    cosf = np.tile(cos, (1, HQ)).astype(np.float32)
    sinf = np.tile(sin, (1, HQ)).astype(np.float32)
    p1 = np.zeros((DH, DH), np.float32)
    for m in range(DH // 2):
        p1[2 * m + 1, 2 * m] = -1.0
        p1[2 * m, 2 * m + 1] = 1.0
    pmat = np.kron(np.eye(HQ, dtype=np.float32), p1)
    return cosf, sinf, pmat


_COSF, _SINF, _PMAT = _rope_consts()


class _Bfly:

    def __init__(self, my, wb, rb, order, send_sems, recv_sems, base):
        self.my = my
        self.wb = wb
        self.rb = rb
        self.order = order
        self.send_sems = send_sems
        self.recv_sems = recv_sems
        self.base = base
        self.lo = jnp.int32(0)
        self.rd = {}
        self.pending = []

    def _mk(self, src, dst, sem_idx, partner):
        return pltpu.make_async_remote_copy(
            src_ref=src, dst_ref=dst,
            send_sem=self.send_sems.at[sem_idx],
            recv_sem=self.recv_sems.at[sem_idx],
            device_id=(partner,),
            device_id_type=pl.DeviceIdType.MESH,
        )

    def rs_start(self, s):
        half = 256 >> s
        mask = self.order[s]
        partner = lax.bitwise_xor(self.my, mask)
        keep_upper = lax.bitwise_and(self.my, mask) != 0
        send_lo = self.lo + jnp.where(keep_upper, 0, half)
        self.lo = self.lo + jnp.where(keep_upper, half, 0)
        r = self._mk(self.wb.at[pl.ds(send_lo, half), :],
                     self.rb.at[pl.ds(ROFFS[s], half), :],
                     self.base + s, partner)
        r.start()
        self.rd[("rs", s)] = r

    def rs_finish(self, s):
        half = 256 >> s
        r = self.rd[("rs", s)]
        r.wait_recv()
        self.pending.append(r)
        seg = self.wb[pl.ds(self.lo, half), :].astype(jnp.float32)
        inc = self.rb[pl.ds(ROFFS[s], half), :].astype(jnp.float32)
        self.wb[pl.ds(self.lo, half), :] = (seg + inc).astype(jnp.bfloat16)

    def ag_start(self, s):
        size = CH << s
        mask = self.order[2 - s]
        partner = lax.bitwise_xor(self.my, mask)
        r = self._mk(self.wb.at[pl.ds(self.lo, size), :],
                     self.wb.at[pl.ds(self.lo, size), :],
                     self.base + 3 + s, partner)
        r.start()
        self.rd[("ag", s)] = r

    def ag_finish(self, s):
        size = CH << s
        mask = self.order[2 - s]
        r = self.rd[("ag", s)]
        r.wait_recv()
        self.pending.append(r)
        self.lo = self.lo - jnp.where(
            lax.bitwise_and(self.my, mask) != 0, size, 0)

    def drain(self):
        for r in self.pending:
            r.wait_send()


def kernel(x, Wq, Wk, Wv, Wo):
    xb = x.reshape(SQ, D).astype(jnp.bfloat16)
    wq = Wq.astype(jnp.bfloat16)
    wk = Wk.astype(jnp.bfloat16)
    wv = Wv.astype(jnp.bfloat16)
    wo = Wo.astype(jnp.bfloat16)
    cosf = jnp.asarray(_COSF)
    sinf = jnp.asarray(_SINF)
    pmat = jnp.asarray(_PMAT, dtype=jnp.bfloat16)

    def body(x_ref, wq_ref, wk_ref, wv_ref, wo_ref, cos_ref, sin_ref, p_ref,
             out_ref, wTA, wTB, wBA, wBB, rTA, rTB, rBA, rBB,
             send_sems, recv_sems):
        my = lax.axis_index("i")

        xv = x_ref[...]
        p = p_ref[...]

        def proj_rope(w_ref):
            t = jnp.dot(xv, w_ref[...], preferred_element_type=jnp.float32)
            tr = jnp.dot(t.astype(jnp.bfloat16), p,
                         preferred_element_type=jnp.float32)
            return (t * cos_ref[...] + tr * sin_ref[...]).astype(jnp.bfloat16)

        q = proj_rope(wq_ref)
        k = proj_rope(wk_ref)
        v = jnp.dot(xv, wv_ref[...],
                    preferred_element_type=jnp.float32).astype(jnp.bfloat16)
        q = (q.astype(jnp.float32) * SCALE).astype(jnp.bfloat16)

        def head_ctx(h, r0):
            sl = slice(h * DH, (h + 1) * DH)
            s = lax.dot_general(q[r0:r0 + HR, sl], k[:, sl],
                                (((1,), (1,)), ((), ())),
                                preferred_element_type=jnp.float32)
            w = jnp.exp(s)
            denom = jnp.sum(w, axis=-1, keepdims=True)
            ctx = jnp.dot(w.astype(jnp.bfloat16), v[:, sl],
                          preferred_element_type=jnp.float32)
            return (ctx / denom).astype(jnp.bfloat16)

        ORD_A, ORD_B = (4, 2, 1), (1, 4, 2)
        TA = _Bfly(my, wTA, rTA, ORD_A, send_sems, recv_sems, 0)
        TB = _Bfly(my, wTB, rTB, ORD_B, send_sems, recv_sems, 6)
        BA = _Bfly(my, wBA, rBA, ORD_A, send_sems, recv_sems, 12)
        BB = _Bfly(my, wBB, rBB, ORD_B, send_sems, recv_sems, 18)

        ctx_top = jnp.concatenate([head_ctx(h, 0) for h in range(HQ)], axis=1)
        acc_top = jnp.dot(ctx_top, wo_ref[...],
                          preferred_element_type=jnp.float32)
        wTA[...] = acc_top[:, :512].astype(jnp.bfloat16)
        wTB[...] = acc_top[:, 512:].astype(jnp.bfloat16)
        TA.rs_start(0)
        TB.rs_start(0)

        ctx_bot = []
        for h in range(HQ):
            ctx_bot.append(head_ctx(h, HR))
            if h == 1:
                TA.rs_finish(0); TB.rs_finish(0)
                TA.rs_start(1); TB.rs_start(1)
            elif h == 3:
                TA.rs_finish(1); TB.rs_finish(1)
                TA.rs_start(2); TB.rs_start(2)
            elif h == 5:
                TA.rs_finish(2); TB.rs_finish(2)
                TA.ag_start(0); TB.ag_start(0)
            elif h == 7:
                TA.ag_finish(0); TB.ag_finish(0)
                TA.ag_start(1); TB.ag_start(1)
        acc_bot = jnp.dot(jnp.concatenate(ctx_bot, axis=1), wo_ref[...],
                          preferred_element_type=jnp.float32)
        wBA[...] = acc_bot[:, :512].astype(jnp.bfloat16)
        wBB[...] = acc_bot[:, 512:].astype(jnp.bfloat16)

        TA.ag_finish(1); TB.ag_finish(1)
        TA.ag_start(2); TB.ag_start(2)
        BA.rs_start(0); BB.rs_start(0)
        TA.ag_finish(2); TB.ag_finish(2)
        out_ref[0, :HR, :512] = wTA[...].astype(jnp.float32)
        out_ref[0, :HR, 512:] = wTB[...].astype(jnp.float32)

        BA.rs_finish(0); BB.rs_finish(0)
        BA.rs_start(1); BB.rs_start(1)
        BA.rs_finish(1); BB.rs_finish(1)
        BA.rs_start(2); BB.rs_start(2)
        BA.rs_finish(2); BB.rs_finish(2)
        BA.ag_start(0); BB.ag_start(0)
        BA.ag_finish(0); BB.ag_finish(0)
        BA.ag_start(1); BB.ag_start(1)
        BA.ag_finish(1); BB.ag_finish(1)
        BA.ag_start(2); BB.ag_start(2)
        BA.ag_finish(2); BB.ag_finish(2)
        out_ref[0, HR:, :512] = wBA[...].astype(jnp.float32)
        out_ref[0, HR:, 512:] = wBB[...].astype(jnp.float32)

        for bf in (TA, TB, BA, BB):
            bf.drain()

    return pl.pallas_call(
        body,
        out_shape=jax.ShapeDtypeStruct((B, SQ, D), jnp.float32),
        in_specs=[pl.BlockSpec(memory_space=pltpu.VMEM)] * 8,
        out_specs=pl.BlockSpec(memory_space=pltpu.VMEM),
        scratch_shapes=[
            pltpu.VMEM((HR, D // 2), jnp.bfloat16),
            pltpu.VMEM((HR, D // 2), jnp.bfloat16),
            pltpu.VMEM((HR, D // 2), jnp.bfloat16),
            pltpu.VMEM((HR, D // 2), jnp.bfloat16),
            pltpu.VMEM((448, D // 2), jnp.bfloat16),
            pltpu.VMEM((448, D // 2), jnp.bfloat16),
            pltpu.VMEM((448, D // 2), jnp.bfloat16),
            pltpu.VMEM((448, D // 2), jnp.bfloat16),
            pltpu.SemaphoreType.DMA((24,)),
            pltpu.SemaphoreType.DMA((24,)),
        ],
    )(xb, wq, wk, wv, wo, cosf, sinf, pmat)
